# baseline (device time: 26382 ns/iter reference)
import jax
import jax.numpy as jnp
from jax import lax
from jax.experimental import pallas as pl
from jax.experimental.pallas import tpu as pltpu

N_DEV = 8
M = 1024
N = 1024
NSUB = 4
C = N // NSUB

TREES = ((0, 384), (384, 384), (768, 256))
AXORDER = ((0, 1, 2), (1, 2, 0), (2, 0, 1))
N_EX = 15 * NSUB


def kernel(x, w_mat):
    def body(x_ref, w_ref, out_ref, acc_ref, ag_ref, *rest):
        s0_send = rest[0:3]
        s0_recv = rest[3:6]
        s1_send = rest[6:9]
        s1_recv = rest[9:12]
        s2_send = rest[12:15]
        s2_recv = rest[15:18]
        send_sems, recv_sems = rest[18:]

        d = lax.axis_index("i")
        m = d % 4
        base = d - m
        partners = (
            base + (m ^ 1),
            base + (m ^ 3),
            d ^ 4,
        )
        bits = (
            (m ^ (m >> 1)) & 1,
            m >> 1,
            d >> 2,
        )

        barrier_sem = pltpu.get_barrier_semaphore()
        for ax in range(3):
            pl.semaphore_signal(
                barrier_sem, inc=1,
                device_id=(partners[ax],),
                device_id_type=pl.DeviceIdType.MESH,
            )
        pl.semaphore_wait(barrier_sem, 3)

        Hs = [r // 2 for _, r in TREES]
        Qs = [r // 4 for _, r in TREES]
        b1 = [bits[AXORDER[t][0]] for t in range(3)]
        b2 = [bits[AXORDER[t][1]] for t in range(3)]
        h_start = [TREES[t][0] + b1[t] * Hs[t] for t in range(3)]
        q_start = [h_start[t] + b2[t] * Qs[t] for t in range(3)]

        def exchange(i, j, src_ref, dst_ref, ax):
            rdma = pltpu.make_async_remote_copy(
                src_ref=src_ref,
                dst_ref=dst_ref,
                send_sem=send_sems.at[i * NSUB + j],
                recv_sem=recv_sems.at[i * NSUB + j],
                device_id=(partners[ax],),
                device_id_type=pl.DeviceIdType.MESH,
            )
            rdma.start()
            return rdma

        rd = {}
        for j in range(NSUB):
            cs = pl.ds(j * C, C)
            acc_ref[:, cs] = jnp.dot(
                x_ref[...], w_ref[:, cs], preferred_element_type=jnp.float32
            )
            for t, (r0, r) in enumerate(TREES):
                send_start = r0 + (1 - b1[t]) * Hs[t]
                s0_send[t][:, cs] = acc_ref[
                    pl.ds(send_start, Hs[t]), cs
                ].astype(jnp.bfloat16)
                rd[(0, t, j)] = exchange(
                    t, j, s0_send[t].at[:, cs], s0_recv[t].at[:, cs],
                    AXORDER[t][0],
                )

        for j in range(NSUB):
            cs = pl.ds(j * C, C)
            for t in range(3):
                rd[(0, t, j)].wait()
                off = (1 - b2[t]) * Qs[t]
                tmp = acc_ref[
                    pl.ds(h_start[t] + off, Qs[t]), cs
                ] + s0_recv[t][pl.ds(off, Qs[t]), cs].astype(jnp.float32)
                s1_send[t][:, cs] = tmp.astype(jnp.bfloat16)
                rd[(1, t, j)] = exchange(
                    3 + t, j, s1_send[t].at[:, cs], s1_recv[t].at[:, cs],
                    AXORDER[t][1],
                )

        for j in range(NSUB):
            cs = pl.ds(j * C, C)
            for t in range(3):
                rd[(1, t, j)].wait()
                off = b2[t] * Qs[t]
                tmp = (
                    acc_ref[pl.ds(q_start[t], Qs[t]), cs]
                    + s0_recv[t][pl.ds(off, Qs[t]), cs].astype(jnp.float32)
                    + s1_recv[t][:, cs].astype(jnp.float32)
                )
                s2_send[t][:, cs] = tmp.astype(jnp.bfloat16)
                rd[(2, t, j)] = exchange(
                    6 + t, j, s2_send[t].at[:, cs], s2_recv[t].at[:, cs],
                    AXORDER[t][2],
                )
                acc_ref[pl.ds(q_start[t], Qs[t]), cs] = tmp
        for j in range(NSUB):
            cs = pl.ds(j * C, C)
            for t in range(3):
                rd[(2, t, j)].wait()
                z = acc_ref[pl.ds(q_start[t], Qs[t]), cs] + s2_recv[t][
                    :, cs
                ].astype(jnp.float32)
                y = z * jax.nn.sigmoid(z)
                ag_ref[pl.ds(q_start[t], Qs[t]), cs] = y.astype(jnp.bfloat16)
                rd[(3, t, j)] = exchange(
                    9 + t, j,
                    ag_ref.at[pl.ds(q_start[t], Qs[t]), cs],
                    ag_ref.at[pl.ds(q_start[t], Qs[t]), cs],
                    AXORDER[t][1],
                )

        for j in range(NSUB):
            cs = pl.ds(j * C, C)
            for t in range(3):
                rd[(3, t, j)].wait()
                rd[(4, t, j)] = exchange(
                    12 + t, j,
                    ag_ref.at[pl.ds(h_start[t], Hs[t]), cs],
                    ag_ref.at[pl.ds(h_start[t], Hs[t]), cs],
                    AXORDER[t][0],
                )

        for j in range(NSUB):
            cs = pl.ds(j * C, C)
            for t, (r0, r) in enumerate(TREES):
                rd[(4, t, j)].wait()
                out_ref[pl.ds(r0, r), cs] = ag_ref[
                    pl.ds(r0, r), cs
                ].astype(jnp.float32)

    scratch = [pltpu.VMEM((M, N), jnp.float32), pltpu.VMEM((M, N), jnp.bfloat16)]
    scratch += [pltpu.VMEM((r // 2, N), jnp.bfloat16) for _, r in TREES]
    scratch += [pltpu.VMEM((r // 2, N), jnp.bfloat16) for _, r in TREES]
    scratch += [pltpu.VMEM((r // 4, N), jnp.bfloat16) for _, r in TREES]
    scratch += [pltpu.VMEM((r // 4, N), jnp.bfloat16) for _, r in TREES]
    scratch += [pltpu.VMEM((r // 4, N), jnp.bfloat16) for _, r in TREES]
    scratch += [pltpu.VMEM((r // 4, N), jnp.bfloat16) for _, r in TREES]
    scratch += [pltpu.SemaphoreType.DMA((N_EX,)) for _ in range(2)]

    return pl.pallas_call(
        body,
        out_shape=jax.ShapeDtypeStruct((M, N), jnp.float32),
        in_specs=[
            pl.BlockSpec(memory_space=pltpu.VMEM),
            pl.BlockSpec(memory_space=pltpu.VMEM),
        ],
        out_specs=pl.BlockSpec(memory_space=pltpu.VMEM),
        scratch_shapes=scratch,
        compiler_params=pltpu.CompilerParams(collective_id=0),
    )(x, w_mat)


# device time: 25782 ns/iter; 1.0233x vs baseline; 1.0233x over previous
import jax
import jax.numpy as jnp
from jax import lax
from jax.experimental import pallas as pl
from jax.experimental.pallas import tpu as pltpu

N_DEV = 8
M = 1024
N = 1024
NSUB = 4
C = N // NSUB

TREES = ((0, 352), (352, 352), (704, 320))
AXORDER = ((0, 1, 2), (1, 2, 0), (2, 0, 1))
N_EX = 12 * NSUB


def kernel(x, w_mat):
    def body(x_ref, w_ref, out_ref, acc_ref, ag_ref, *rest):
        send_bufs = rest[0:9]
        recv_bufs = rest[9:18]
        send_sems, recv_sems = rest[18:]

        d = lax.axis_index("i")
        m = d % 4
        base = d - m
        partners = (
            base + (m ^ 1),
            base + (m ^ 3),
            d ^ 4,
        )
        bits = (
            (m ^ (m >> 1)) & 1,
            m >> 1,
            d >> 2,
        )

        barrier_sem = pltpu.get_barrier_semaphore()
        for ax in range(3):
            pl.semaphore_signal(
                barrier_sem, inc=1,
                device_id=(partners[ax],),
                device_id_type=pl.DeviceIdType.MESH,
            )
        pl.semaphore_wait(barrier_sem, 3)

        halves = [r // 2 for _, r in TREES]
        keeps = [r0 + bits[AXORDER[t][0]] * (r // 2)
                 for t, (r0, r) in enumerate(TREES)]

        def exchange(i, j, src_ref, dst_ref, ax):
            rdma = pltpu.make_async_remote_copy(
                src_ref=src_ref,
                dst_ref=dst_ref,
                send_sem=send_sems.at[i * NSUB + j],
                recv_sem=recv_sems.at[i * NSUB + j],
                device_id=(partners[ax],),
                device_id_type=pl.DeviceIdType.MESH,
            )
            rdma.start()
            return rdma

        rd = {}
        for j in range(NSUB):
            cs = pl.ds(j * C, C)
            acc_ref[:, cs] = jnp.dot(
                x_ref[...], w_ref[:, cs], preferred_element_type=jnp.float32
            )
            for t, (r0, r) in enumerate(TREES):
                h = halves[t]
                b = bits[AXORDER[t][0]]
                send_start = r0 + (1 - b) * h
                i = t * 3
                send_bufs[i][:, cs] = acc_ref[
                    pl.ds(send_start, h), cs
                ].astype(jnp.bfloat16)
                rd[(0, t, j)] = exchange(
                    i, j, send_bufs[i].at[:, cs], recv_bufs[i].at[:, cs],
                    AXORDER[t][0],
                )

        for s in (1, 2):
            for j in range(NSUB):
                cs = pl.ds(j * C, C)
                for t in range(3):
                    h = halves[t]
                    i = t * 3 + s
                    rd[(s - 1, t, j)].wait()
                    tmp = acc_ref[pl.ds(keeps[t], h), cs] + recv_bufs[
                        i - 1
                    ][:, cs].astype(jnp.float32)
                    send_bufs[i][:, cs] = tmp.astype(jnp.bfloat16)
                    rd[(s, t, j)] = exchange(
                        i, j, send_bufs[i].at[:, cs], recv_bufs[i].at[:, cs],
                        AXORDER[t][s],
                    )
                    acc_ref[pl.ds(keeps[t], h), cs] = tmp

        for j in range(NSUB):
            cs = pl.ds(j * C, C)
            for t in range(3):
                h = halves[t]
                rd[(2, t, j)].wait()
                z = acc_ref[pl.ds(keeps[t], h), cs] + recv_bufs[
                    t * 3 + 2
                ][:, cs].astype(jnp.float32)
                y = z * jax.nn.sigmoid(z)
                ag_ref[pl.ds(keeps[t], h), cs] = y.astype(jnp.bfloat16)
                rd[(3, t, j)] = exchange(
                    9 + t, j,
                    ag_ref.at[pl.ds(keeps[t], h), cs],
                    ag_ref.at[pl.ds(keeps[t], h), cs],
                    AXORDER[t][0],
                )

        for j in range(NSUB):
            cs = pl.ds(j * C, C)
            for t, (r0, r) in enumerate(TREES):
                rd[(3, t, j)].wait()
                out_ref[pl.ds(r0, r), cs] = ag_ref[
                    pl.ds(r0, r), cs
                ].astype(jnp.float32)

    scratch = [pltpu.VMEM((M, N), jnp.float32), pltpu.VMEM((M, N), jnp.bfloat16)]
    for _, r in TREES:
        scratch += [pltpu.VMEM((r // 2, N), jnp.bfloat16)] * 3
    for _, r in TREES:
        scratch += [pltpu.VMEM((r // 2, N), jnp.bfloat16)] * 3
    scratch += [pltpu.SemaphoreType.DMA((N_EX,)) for _ in range(2)]

    return pl.pallas_call(
        body,
        out_shape=jax.ShapeDtypeStruct((M, N), jnp.float32),
        in_specs=[
            pl.BlockSpec(memory_space=pltpu.VMEM),
            pl.BlockSpec(memory_space=pltpu.VMEM),
        ],
        out_specs=pl.BlockSpec(memory_space=pltpu.VMEM),
        scratch_shapes=scratch,
        compiler_params=pltpu.CompilerParams(collective_id=0),
    )(x, w_mat)


# device time: 25106 ns/iter; 1.0508x vs baseline; 1.0269x over previous
import jax
import jax.numpy as jnp
from jax import lax
from jax.experimental import pallas as pl
from jax.experimental.pallas import tpu as pltpu

N_DEV = 8
M = 1024
N = 1024
NSUB = 4
C = N // NSUB

TREES = ((0, 352), (352, 352), (704, 320))
AXORDER = ((0, 1, 2), (1, 2, 0), (2, 0, 1))
N_EX = 12 * NSUB


def kernel(x, w_mat):
    def body(x_ref, w_ref, out_ref, acc_ref, *rest):
        ag_ref = out_ref
        send_bufs = rest[0:9]
        recv_bufs = rest[9:18]
        send_sems, recv_sems = rest[18:]

        d = lax.axis_index("i")
        m = d % 4
        base = d - m
        partners = (
            base + (m ^ 1),
            base + (m ^ 3),
            d ^ 4,
        )
        bits = (
            (m ^ (m >> 1)) & 1,
            m >> 1,
            d >> 2,
        )

        barrier_sem = pltpu.get_barrier_semaphore()
        for ax in range(3):
            pl.semaphore_signal(
                barrier_sem, inc=1,
                device_id=(partners[ax],),
                device_id_type=pl.DeviceIdType.MESH,
            )
        pl.semaphore_wait(barrier_sem, 3)

        halves = [r // 2 for _, r in TREES]
        keeps = [r0 + bits[AXORDER[t][0]] * (r // 2)
                 for t, (r0, r) in enumerate(TREES)]

        def exchange(i, j, src_ref, dst_ref, ax):
            rdma = pltpu.make_async_remote_copy(
                src_ref=src_ref,
                dst_ref=dst_ref,
                send_sem=send_sems.at[i * NSUB + j],
                recv_sem=recv_sems.at[i * NSUB + j],
                device_id=(partners[ax],),
                device_id_type=pl.DeviceIdType.MESH,
            )
            rdma.start()
            return rdma

        rd = {}
        for j in range(NSUB):
            cs = pl.ds(j * C, C)
            acc_ref[:, cs] = jnp.dot(
                x_ref[...], w_ref[:, cs], preferred_element_type=jnp.float32
            )
            for t, (r0, r) in enumerate(TREES):
                h = halves[t]
                b = bits[AXORDER[t][0]]
                send_start = r0 + (1 - b) * h
                i = t * 3
                send_bufs[i][:, cs] = acc_ref[
                    pl.ds(send_start, h), cs
                ].astype(jnp.bfloat16)
                rd[(0, t, j)] = exchange(
                    i, j, send_bufs[i].at[:, cs], recv_bufs[i].at[:, cs],
                    AXORDER[t][0],
                )

        for s in (1, 2):
            for j in range(NSUB):
                cs = pl.ds(j * C, C)
                for t in range(3):
                    h = halves[t]
                    i = t * 3 + s
                    rd[(s - 1, t, j)].wait()
                    tmp = acc_ref[pl.ds(keeps[t], h), cs] + recv_bufs[
                        i - 1
                    ][:, cs].astype(jnp.float32)
                    send_bufs[i][:, cs] = tmp.astype(jnp.bfloat16)
                    rd[(s, t, j)] = exchange(
                        i, j, send_bufs[i].at[:, cs], recv_bufs[i].at[:, cs],
                        AXORDER[t][s],
                    )
                    acc_ref[pl.ds(keeps[t], h), cs] = tmp

        for j in range(NSUB):
            cs = pl.ds(j * C, C)
            for t in range(3):
                h = halves[t]
                rd[(2, t, j)].wait()
                z = acc_ref[pl.ds(keeps[t], h), cs] + recv_bufs[
                    t * 3 + 2
                ][:, cs].astype(jnp.float32)
                y = z * jax.nn.sigmoid(z)
                ag_ref[pl.ds(keeps[t], h), cs] = y.astype(jnp.bfloat16)
                rd[(3, t, j)] = exchange(
                    9 + t, j,
                    ag_ref.at[pl.ds(keeps[t], h), cs],
                    ag_ref.at[pl.ds(keeps[t], h), cs],
                    AXORDER[t][0],
                )

        for j in range(NSUB):
            for t in range(3):
                rd[(3, t, j)].wait()

    scratch = [pltpu.VMEM((M, N), jnp.float32)]
    for _, r in TREES:
        scratch += [pltpu.VMEM((r // 2, N), jnp.bfloat16)] * 3
    for _, r in TREES:
        scratch += [pltpu.VMEM((r // 2, N), jnp.bfloat16)] * 3
    scratch += [pltpu.SemaphoreType.DMA((N_EX,)) for _ in range(2)]

    return pl.pallas_call(
        body,
        out_shape=jax.ShapeDtypeStruct((M, N), jnp.bfloat16),
        in_specs=[
            pl.BlockSpec(memory_space=pltpu.VMEM),
            pl.BlockSpec(memory_space=pltpu.VMEM),
        ],
        out_specs=pl.BlockSpec(memory_space=pltpu.VMEM),
        scratch_shapes=scratch,
        compiler_params=pltpu.CompilerParams(collective_id=0),
    )(x, w_mat)
